# Initial kernel scaffold; baseline (speedup 1.0000x reference)
import jax
import jax.numpy as jnp
from jax.experimental import pallas as pl


def kernel(category_ids, color_ids, style_ids, cat_table, color_table, style_table, W, b):
    raise NotImplementedError("write your pallas kernel here")


# speedup vs baseline: 4.1136x; 4.1136x over previous
import functools

import jax
import jax.numpy as jnp
from jax import lax
from jax.experimental import pallas as pl
from jax.experimental.pallas import tpu as pltpu
from jax.experimental.pallas import tpu_sc as plsc

B, L, D = 4096, 50, 128
CAT_V, COL_V, STY_V = 100000, 1000, 1000
T = B * L

ROWS_PER_BLK = 4000
N_CAT_BLKS = CAT_V // ROWS_PER_BLK
FUSED_ROWS = (N_CAT_BLKS + 1) * ROWS_PER_BLK
COL_BASE = CAT_V
STY_BASE = CAT_V + COL_V

NC, NS = 2, 16
NW = NC * NS
PER_W = T // NW
CHUNK = 128
N_CHUNKS = PER_W // CHUNK


def _build_body(cat_ref, col_ref, sty_ref, w_ref, b_ref, out_ref):
    i = pl.program_id(0)

    @pl.when(i < N_CAT_BLKS)
    def _():
        out_ref[...] = jnp.dot(cat_ref[...], w_ref[0:D, :],
                               preferred_element_type=jnp.float32)

    @pl.when(i == N_CAT_BLKS)
    def _():
        out_ref[0:COL_V, :] = (
            jnp.dot(col_ref[...], w_ref[D:2 * D, :],
                    preferred_element_type=jnp.float32) + b_ref[...])
        out_ref[COL_V:COL_V + STY_V, :] = jnp.dot(
            sty_ref[...], w_ref[2 * D:3 * D, :],
            preferred_element_type=jnp.float32)
        out_ref[COL_V + STY_V:ROWS_PER_BLK, :] = jnp.zeros(
            (ROWS_PER_BLK - COL_V - STY_V, D), jnp.float32)


def _build_fused_table(cat_table, color_table, style_table, W, b2d):
    return pl.pallas_call(
        _build_body,
        grid=(N_CAT_BLKS + 1,),
        in_specs=[
            pl.BlockSpec((ROWS_PER_BLK, D),
                         lambda i: (jnp.minimum(i, N_CAT_BLKS - 1), 0)),
            pl.BlockSpec((COL_V, D), lambda i: (0, 0)),
            pl.BlockSpec((STY_V, D), lambda i: (0, 0)),
            pl.BlockSpec((3 * D, D), lambda i: (0, 0)),
            pl.BlockSpec((1, D), lambda i: (0, 0)),
        ],
        out_specs=pl.BlockSpec((ROWS_PER_BLK, D), lambda i: (i, 0)),
        out_shape=jax.ShapeDtypeStruct((FUSED_ROWS, D), jnp.float32),
    )(cat_table, color_table, style_table, W, b2d)


def _sc_body(ftab, cid, colid, styid, out,
             idxa, idxb, idxc, bufa, bufb, bufc, sema, semb, semc):
    wid = lax.axis_index("s") * NC + lax.axis_index("c")
    w_base = wid * PER_W

    def chunk(g, carry):
        base = w_base + g * CHUNK
        pltpu.sync_copy(cid.at[pl.ds(base, CHUNK)], idxa)
        pltpu.sync_copy(colid.at[pl.ds(base, CHUNK)], idxb)
        pltpu.sync_copy(styid.at[pl.ds(base, CHUNK)], idxc)

        def rebase(k, c2):
            sl = pl.ds(k * 16, 16)
            idxb[sl] = idxb[sl] + COL_BASE
            idxc[sl] = idxc[sl] + STY_BASE
            return c2
        lax.fori_loop(0, CHUNK // 16, rebase, 0)

        ca = pltpu.async_copy(ftab.at[idxa], bufa, sema)
        cb = pltpu.async_copy(ftab.at[idxb], bufb, semb)
        cc = pltpu.async_copy(ftab.at[idxc], bufc, semc)
        ca.wait()
        cb.wait()
        cc.wait()

        def tok(t, c2):
            for d in range(D // 16):
                sl = pl.ds(d * 16, 16)
                v = bufa[t, sl] + bufb[t, sl] + bufc[t, sl]
                bufa[t, sl] = jnp.maximum(v, 0.0)
            return c2
        lax.fori_loop(0, CHUNK, tok, 0)

        pltpu.sync_copy(bufa, out.at[pl.ds(base, CHUNK)])
        return carry

    lax.fori_loop(0, N_CHUNKS, chunk, 0)


def _sc_fuse(ftab, cid, colid, styid):
    mesh = plsc.VectorSubcoreMesh(core_axis_name="c", subcore_axis_name="s")
    fn = functools.partial(
        pl.kernel,
        mesh=mesh,
        out_type=jax.ShapeDtypeStruct((T, D), jnp.float32),
        scratch_types=[
            pltpu.VMEM((CHUNK,), jnp.int32),
            pltpu.VMEM((CHUNK,), jnp.int32),
            pltpu.VMEM((CHUNK,), jnp.int32),
            pltpu.VMEM((CHUNK, D), jnp.float32),
            pltpu.VMEM((CHUNK, D), jnp.float32),
            pltpu.VMEM((CHUNK, D), jnp.float32),
            pltpu.SemaphoreType.DMA,
            pltpu.SemaphoreType.DMA,
            pltpu.SemaphoreType.DMA,
        ],
    )(_sc_body)
    return fn(ftab, cid, colid, styid)


def kernel(category_ids, color_ids, style_ids, cat_table, color_table,
           style_table, W, b):
    ftab = _build_fused_table(cat_table, color_table, style_table, W,
                              b.reshape(1, D))
    out = _sc_fuse(ftab,
                   category_ids.reshape(-1),
                   color_ids.reshape(-1),
                   style_ids.reshape(-1))
    return out.reshape(B, L, D)


# speedup vs baseline: 5.5301x; 1.3443x over previous
import functools

import jax
import jax.numpy as jnp
from jax import lax
from jax.experimental import pallas as pl
from jax.experimental.pallas import tpu as pltpu
from jax.experimental.pallas import tpu_sc as plsc

B, L, D = 4096, 50, 128
CAT_V, COL_V, STY_V = 100000, 1000, 1000
T = B * L

ROWS_PER_BLK = 4000
N_CAT_BLKS = CAT_V // ROWS_PER_BLK
FUSED_ROWS = (N_CAT_BLKS + 1) * ROWS_PER_BLK
COL_BASE = CAT_V
STY_BASE = CAT_V + COL_V

NC, NS = 2, 16
NW = NC * NS
PER_W = T // NW
CHUNK = 128
N_CHUNKS = PER_W // CHUNK


def _build_body(cat_ref, col_ref, sty_ref, w_ref, b_ref, out_ref):
    i = pl.program_id(0)

    @pl.when(i < N_CAT_BLKS)
    def _():
        out_ref[...] = jnp.dot(cat_ref[...], w_ref[0:D, :],
                               preferred_element_type=jnp.float32)

    @pl.when(i == N_CAT_BLKS)
    def _():
        out_ref[0:COL_V, :] = (
            jnp.dot(col_ref[...], w_ref[D:2 * D, :],
                    preferred_element_type=jnp.float32) + b_ref[...])
        out_ref[COL_V:COL_V + STY_V, :] = jnp.dot(
            sty_ref[...], w_ref[2 * D:3 * D, :],
            preferred_element_type=jnp.float32)
        out_ref[COL_V + STY_V:ROWS_PER_BLK, :] = jnp.zeros(
            (ROWS_PER_BLK - COL_V - STY_V, D), jnp.float32)


def _build_fused_table(cat_table, color_table, style_table, W, b2d):
    return pl.pallas_call(
        _build_body,
        grid=(N_CAT_BLKS + 1,),
        in_specs=[
            pl.BlockSpec((ROWS_PER_BLK, D),
                         lambda i: (jnp.minimum(i, N_CAT_BLKS - 1), 0)),
            pl.BlockSpec((COL_V, D), lambda i: (0, 0)),
            pl.BlockSpec((STY_V, D), lambda i: (0, 0)),
            pl.BlockSpec((3 * D, D), lambda i: (0, 0)),
            pl.BlockSpec((1, D), lambda i: (0, 0)),
        ],
        out_specs=pl.BlockSpec((ROWS_PER_BLK, D), lambda i: (i, 0)),
        out_shape=jax.ShapeDtypeStruct((FUSED_ROWS, D), jnp.float32),
    )(cat_table, color_table, style_table, W, b2d)


def _sc_body(ftab, cid, colid, styid, out,
             idxa, idxb, idxc,
             buf00, buf01, buf02, buf10, buf11, buf12,
             gsem0, gsem1, osem0, osem1):
    wid = lax.axis_index("s") * NC + lax.axis_index("c")
    w_base = wid * PER_W

    pltpu.sync_copy(cid.at[pl.ds(w_base, PER_W)], idxa)
    pltpu.sync_copy(colid.at[pl.ds(w_base, PER_W)], idxb)
    pltpu.sync_copy(styid.at[pl.ds(w_base, PER_W)], idxc)

    def rebase(k, c2):
        sl = pl.ds(k * 16, 16)
        idxb[sl] = idxb[sl] + COL_BASE
        idxc[sl] = idxc[sl] + STY_BASE
        return c2
    lax.fori_loop(0, PER_W // 16, rebase, 0)

    bufs = ((buf00, buf01, buf02), (buf10, buf11, buf12))
    gsems = (gsem0, gsem1)
    osems = (osem0, osem1)

    def gather_copies(g, k):
        off = g * CHUNK
        return (
            pltpu.make_async_copy(ftab.at[idxa.at[pl.ds(off, CHUNK)]],
                                  bufs[k][0], gsems[k]),
            pltpu.make_async_copy(ftab.at[idxb.at[pl.ds(off, CHUNK)]],
                                  bufs[k][1], gsems[k]),
            pltpu.make_async_copy(ftab.at[idxc.at[pl.ds(off, CHUNK)]],
                                  bufs[k][2], gsems[k]),
        )

    def fire(g, k):
        for c in gather_copies(g, k):
            c.start()

    def wait_gathers(g, k):
        for c in gather_copies(g, k):
            c.wait()

    def compute(k):
        ba, bb, bc = bufs[k]

        def tok(t, c2):
            for d in range(D // 16):
                sl = pl.ds(d * 16, 16)
                v = ba[t, sl] + bb[t, sl] + bc[t, sl]
                ba[t, sl] = jnp.maximum(v, 0.0)
            return c2
        lax.fori_loop(0, CHUNK, tok, 0)

    def put_copy(g, k):
        return pltpu.make_async_copy(
            bufs[k][0], out.at[pl.ds(w_base + g * CHUNK, CHUNK)], osems[k])

    def step(i, carry):
        g0 = 2 * i
        g1 = g0 + 1

        @pl.when(i > 0)
        def _():
            put_copy(g1 - 2, 1).wait()
        fire(g1, 1)

        wait_gathers(g0, 0)
        compute(0)
        put_copy(g0, 0).start()
        put_copy(g0, 0).wait()

        @pl.when(i < N_CHUNKS // 2 - 1)
        def _():
            fire(g0 + 2, 0)

        wait_gathers(g1, 1)
        compute(1)
        put_copy(g1, 1).start()
        return carry

    fire(0, 0)
    lax.fori_loop(0, N_CHUNKS // 2, step, 0)
    put_copy(N_CHUNKS - 1, 1).wait()


def _sc_fuse(ftab, cid, colid, styid):
    mesh = plsc.VectorSubcoreMesh(core_axis_name="c", subcore_axis_name="s")
    fn = functools.partial(
        pl.kernel,
        mesh=mesh,
        out_type=jax.ShapeDtypeStruct((T, D), jnp.float32),
        scratch_types=[
            pltpu.VMEM((PER_W,), jnp.int32),
            pltpu.VMEM((PER_W,), jnp.int32),
            pltpu.VMEM((PER_W,), jnp.int32),
            pltpu.VMEM((CHUNK, D), jnp.float32),
            pltpu.VMEM((CHUNK, D), jnp.float32),
            pltpu.VMEM((CHUNK, D), jnp.float32),
            pltpu.VMEM((CHUNK, D), jnp.float32),
            pltpu.VMEM((CHUNK, D), jnp.float32),
            pltpu.VMEM((CHUNK, D), jnp.float32),
            pltpu.SemaphoreType.DMA,
            pltpu.SemaphoreType.DMA,
            pltpu.SemaphoreType.DMA,
            pltpu.SemaphoreType.DMA,
        ],
    )(_sc_body)
    return fn(ftab, cid, colid, styid)


def kernel(category_ids, color_ids, style_ids, cat_table, color_table,
           style_table, W, b):
    ftab = _build_fused_table(cat_table, color_table, style_table, W,
                              b.reshape(1, D))
    out = _sc_fuse(ftab,
                   category_ids.reshape(-1),
                   color_ids.reshape(-1),
                   style_ids.reshape(-1))
    return out.reshape(B, L, D)


# speedup vs baseline: 7.8825x; 1.4254x over previous
import functools

import jax
import jax.numpy as jnp
from jax import lax
from jax.experimental import pallas as pl
from jax.experimental.pallas import tpu as pltpu
from jax.experimental.pallas import tpu_sc as plsc

B, L, D = 4096, 50, 128
CAT_V, COL_V, STY_V = 100000, 1000, 1000
T = B * L

ROWS_PER_BLK = 4000
N_CAT_BLKS = CAT_V // ROWS_PER_BLK
FUSED_ROWS = (N_CAT_BLKS + 1) * ROWS_PER_BLK
COL_BASE = CAT_V
STY_BASE = CAT_V + COL_V

NC, NS = 2, 16
NW = NC * NS
BATCH_PER_W = B // NW
CHUNK_B = 2
CHUNK = CHUNK_B * L
N_CHUNKS = BATCH_PER_W // CHUNK_B
IDX_PAD = 104
N_ROWS = T // CHUNK


def _build_body(cat_ref, col_ref, sty_ref, w_ref, b_ref, out_ref):
    i = pl.program_id(0)

    @pl.when(i < N_CAT_BLKS)
    def _():
        out_ref[...] = jnp.dot(cat_ref[...], w_ref[0:D, :],
                               preferred_element_type=jnp.float32)

    @pl.when(i == N_CAT_BLKS)
    def _():
        out_ref[0:COL_V, :] = (
            jnp.dot(col_ref[...], w_ref[D:2 * D, :],
                    preferred_element_type=jnp.float32) + b_ref[...])
        out_ref[COL_V:COL_V + STY_V, :] = jnp.dot(
            sty_ref[...], w_ref[2 * D:3 * D, :],
            preferred_element_type=jnp.float32)
        out_ref[COL_V + STY_V:ROWS_PER_BLK, :] = jnp.zeros(
            (ROWS_PER_BLK - COL_V - STY_V, D), jnp.float32)


def _build_fused_table(cat_table, color_table, style_table, W, b2d):
    return pl.pallas_call(
        _build_body,
        grid=(N_CAT_BLKS + 1,),
        in_specs=[
            pl.BlockSpec((ROWS_PER_BLK, D),
                         lambda i: (jnp.minimum(i, N_CAT_BLKS - 1), 0)),
            pl.BlockSpec((COL_V, D), lambda i: (0, 0)),
            pl.BlockSpec((STY_V, D), lambda i: (0, 0)),
            pl.BlockSpec((3 * D, D), lambda i: (0, 0)),
            pl.BlockSpec((1, D), lambda i: (0, 0)),
        ],
        out_specs=pl.BlockSpec((ROWS_PER_BLK, D), lambda i: (i, 0)),
        out_shape=jax.ShapeDtypeStruct((FUSED_ROWS, D), jnp.float32),
    )(cat_table, color_table, style_table, W, b2d)


def _sc_body(ftab, cid, colid, styid, out,
             idx00, idx01, idx02, idx10, idx11, idx12,
             buf0, buf1, isem0, isem1, gsem0, gsem1, osem0, osem1):
    wid = lax.axis_index("s") * NC + lax.axis_index("c")
    w_row = wid * N_CHUNKS
    w_batch = wid * BATCH_PER_W

    idxs = ((idx00, idx01, idx02), (idx10, idx11, idx12))
    bufs = (buf0, buf1)
    isems = (isem0, isem1)
    gsems = (gsem0, gsem1)
    osems = (osem0, osem1)

    def idx_copies(g, k):
        row = w_row + g
        return (
            pltpu.make_async_copy(cid.at[row], idxs[k][0], isems[k]),
            pltpu.make_async_copy(colid.at[row], idxs[k][1], isems[k]),
            pltpu.make_async_copy(styid.at[row], idxs[k][2], isems[k]),
        )

    def gather_copies(g, k):
        return (
            pltpu.make_async_copy(ftab.at[idxs[k][0].at[pl.ds(0, CHUNK)]],
                                  bufs[k].at[pl.ds(0, CHUNK)], gsems[k]),
            pltpu.make_async_copy(ftab.at[idxs[k][1].at[pl.ds(0, CHUNK)]],
                                  bufs[k].at[pl.ds(CHUNK, CHUNK)], gsems[k]),
            pltpu.make_async_copy(ftab.at[idxs[k][2].at[pl.ds(0, CHUNK)]],
                                  bufs[k].at[pl.ds(2 * CHUNK, CHUNK)],
                                  gsems[k]),
        )

    def fire_idx(g, k):
        for c in idx_copies(g, k):
            c.start()

    def fire_gathers(g, k):
        for c in idx_copies(g, k):
            c.wait()
        for c in gather_copies(g, k):
            c.start()

    def wait_gathers(g, k):
        for c in gather_copies(g, k):
            c.wait()

    def compute(k):
        buf = bufs[k]

        def tok(t, c2):
            for c in range(D // 16):
                sl = pl.ds(c * 16, 16)
                v = buf[t, sl] + buf[CHUNK + t, sl] + buf[2 * CHUNK + t, sl]
                buf[t, sl] = jnp.maximum(v, 0.0)
            return c2
        lax.fori_loop(0, CHUNK, tok, 0)

    def put_copies(g, k):
        b0 = w_batch + CHUNK_B * g
        return tuple(
            pltpu.make_async_copy(bufs[k].at[pl.ds(j * L, L)],
                                  out.at[b0 + j], osems[k])
            for j in range(CHUNK_B))

    def put(g, k):
        for c in put_copies(g, k):
            c.start()

    def wait_put(g, k):
        for c in put_copies(g, k):
            c.wait()

    def step(i, carry):
        g0 = 2 * i
        g1 = g0 + 1
        not_last = i < N_CHUNKS // 2 - 1

        @pl.when(i > 0)
        def _():
            wait_put(g1 - 2, 1)
        fire_gathers(g1, 1)

        wait_gathers(g0, 0)

        @pl.when(not_last)
        def _():
            fire_idx(g0 + 2, 0)
        compute(0)
        put(g0, 0)
        wait_put(g0, 0)

        @pl.when(not_last)
        def _():
            fire_gathers(g0 + 2, 0)

        wait_gathers(g1, 1)

        @pl.when(not_last)
        def _():
            fire_idx(g1 + 2, 1)
        compute(1)
        put(g1, 1)
        return carry

    fire_idx(0, 0)
    fire_gathers(0, 0)
    fire_idx(1, 1)
    lax.fori_loop(0, N_CHUNKS // 2, step, 0)
    wait_put(N_CHUNKS - 1, 1)


def _sc_fuse(ftab, cid, colid, styid):
    mesh = plsc.VectorSubcoreMesh(core_axis_name="c", subcore_axis_name="s")
    fn = functools.partial(
        pl.kernel,
        mesh=mesh,
        out_type=jax.ShapeDtypeStruct((B, L, D), jnp.float32),
        scratch_types=[
            pltpu.VMEM((IDX_PAD,), jnp.int32),
            pltpu.VMEM((IDX_PAD,), jnp.int32),
            pltpu.VMEM((IDX_PAD,), jnp.int32),
            pltpu.VMEM((IDX_PAD,), jnp.int32),
            pltpu.VMEM((IDX_PAD,), jnp.int32),
            pltpu.VMEM((IDX_PAD,), jnp.int32),
            pltpu.VMEM((3 * CHUNK, D), jnp.float32),
            pltpu.VMEM((3 * CHUNK, D), jnp.float32),
            pltpu.SemaphoreType.DMA,
            pltpu.SemaphoreType.DMA,
            pltpu.SemaphoreType.DMA,
            pltpu.SemaphoreType.DMA,
            pltpu.SemaphoreType.DMA,
            pltpu.SemaphoreType.DMA,
        ],
    )(_sc_body)
    return fn(ftab, cid, colid, styid)


def _pad_ids(ids, base):
    rows = ids.reshape(N_ROWS, CHUNK).astype(jnp.int32) + base
    return jnp.pad(rows, ((0, 0), (0, IDX_PAD - CHUNK)))


def kernel(category_ids, color_ids, style_ids, cat_table, color_table,
           style_table, W, b):
    ftab = _build_fused_table(cat_table, color_table, style_table, W,
                              b.reshape(1, D))
    return _sc_fuse(ftab,
                    _pad_ids(category_ids, 0),
                    _pad_ids(color_ids, COL_BASE),
                    _pad_ids(style_ids, STY_BASE))


# speedup vs baseline: 10.4643x; 1.3275x over previous
import functools

import jax
import jax.numpy as jnp
from jax import lax
from jax.experimental import pallas as pl
from jax.experimental.pallas import tpu as pltpu
from jax.experimental.pallas import tpu_sc as plsc

B, L, D = 4096, 50, 128
CAT_V, COL_V, STY_V = 100000, 1000, 1000
T = B * L

ROWS_PER_BLK = 4000
N_CAT_BLKS = CAT_V // ROWS_PER_BLK
FUSED_ROWS = (N_CAT_BLKS + 1) * ROWS_PER_BLK
COL_BASE = CAT_V
STY_BASE = CAT_V + COL_V

NC, NS = 2, 16
NW = NC * NS
BATCH_PER_W = B // NW
CHUNK = BATCH_PER_W
N_CHUNKS = L


def _build_body(cat_ref, col_ref, sty_ref, w_ref, b_ref, out_ref):
    i = pl.program_id(0)

    @pl.when(i < N_CAT_BLKS)
    def _():
        out_ref[...] = jnp.dot(cat_ref[...], w_ref[0:D, :],
                               preferred_element_type=jnp.float32)

    @pl.when(i == N_CAT_BLKS)
    def _():
        out_ref[0:COL_V, :] = (
            jnp.dot(col_ref[...], w_ref[D:2 * D, :],
                    preferred_element_type=jnp.float32) + b_ref[...])
        out_ref[COL_V:COL_V + STY_V, :] = jnp.dot(
            sty_ref[...], w_ref[2 * D:3 * D, :],
            preferred_element_type=jnp.float32)
        out_ref[COL_V + STY_V:ROWS_PER_BLK, :] = jnp.zeros(
            (ROWS_PER_BLK - COL_V - STY_V, D), jnp.float32)


def _build_fused_table(cat_table, color_table, style_table, W, b2d):
    return pl.pallas_call(
        _build_body,
        grid=(N_CAT_BLKS + 1,),
        in_specs=[
            pl.BlockSpec((ROWS_PER_BLK, D),
                         lambda i: (jnp.minimum(i, N_CAT_BLKS - 1), 0)),
            pl.BlockSpec((COL_V, D), lambda i: (0, 0)),
            pl.BlockSpec((STY_V, D), lambda i: (0, 0)),
            pl.BlockSpec((3 * D, D), lambda i: (0, 0)),
            pl.BlockSpec((1, D), lambda i: (0, 0)),
        ],
        out_specs=pl.BlockSpec((ROWS_PER_BLK, D), lambda i: (i, 0)),
        out_shape=jax.ShapeDtypeStruct((FUSED_ROWS, D), jnp.float32),
    )(cat_table, color_table, style_table, W, b2d)


def _sc_body(ftab, cid, colid, styid, out,
             idx00, idx01, idx02, idx10, idx11, idx12,
             buf0, buf1, isem0, isem1, gsem0, gsem1, osem0, osem1):
    wid = lax.axis_index("s") * NC + lax.axis_index("c")
    wb = wid * BATCH_PER_W

    idxs = ((idx00, idx01, idx02), (idx10, idx11, idx12))
    bufs = (buf0, buf1)
    isems = (isem0, isem1)
    gsems = (gsem0, gsem1)
    osems = (osem0, osem1)

    def idx_copies(g, k):
        return (
            pltpu.make_async_copy(cid.at[g, pl.ds(wb, CHUNK)],
                                  idxs[k][0], isems[k]),
            pltpu.make_async_copy(colid.at[g, pl.ds(wb, CHUNK)],
                                  idxs[k][1], isems[k]),
            pltpu.make_async_copy(styid.at[g, pl.ds(wb, CHUNK)],
                                  idxs[k][2], isems[k]),
        )

    def gather_copies(g, k):
        return (
            pltpu.make_async_copy(ftab.at[idxs[k][0]],
                                  bufs[k].at[pl.ds(0, CHUNK)], gsems[k]),
            pltpu.make_async_copy(ftab.at[idxs[k][1]],
                                  bufs[k].at[pl.ds(CHUNK, CHUNK)], gsems[k]),
            pltpu.make_async_copy(ftab.at[idxs[k][2]],
                                  bufs[k].at[pl.ds(2 * CHUNK, CHUNK)],
                                  gsems[k]),
        )

    def fire_idx(g, k):
        for c in idx_copies(g, k):
            c.start()

    def fire_gathers(g, k):
        for c in idx_copies(g, k):
            c.wait()
        for c in gather_copies(g, k):
            c.start()

    def wait_gathers(g, k):
        for c in gather_copies(g, k):
            c.wait()

    def compute(k):
        buf = bufs[k]

        def tok(t, c2):
            for c in range(D // 16):
                sl = pl.ds(c * 16, 16)
                v = buf[t, sl] + buf[CHUNK + t, sl] + buf[2 * CHUNK + t, sl]
                buf[t, sl] = jnp.maximum(v, 0.0)
            return c2
        lax.fori_loop(0, CHUNK, tok, 0)

    def put_copy(g, k):
        return pltpu.make_async_copy(bufs[k].at[pl.ds(0, CHUNK)],
                                     out.at[g, pl.ds(wb, CHUNK)], osems[k])

    def step(i, carry):
        g0 = 2 * i
        g1 = g0 + 1
        not_last = i < N_CHUNKS // 2 - 1

        @pl.when(i > 0)
        def _():
            put_copy(g1 - 2, 1).wait()
        fire_gathers(g1, 1)

        wait_gathers(g0, 0)

        @pl.when(not_last)
        def _():
            fire_idx(g0 + 2, 0)
        compute(0)
        put_copy(g0, 0).start()
        put_copy(g0, 0).wait()

        @pl.when(not_last)
        def _():
            fire_gathers(g0 + 2, 0)

        wait_gathers(g1, 1)

        @pl.when(not_last)
        def _():
            fire_idx(g1 + 2, 1)
        compute(1)
        put_copy(g1, 1).start()
        return carry

    fire_idx(0, 0)
    fire_gathers(0, 0)
    fire_idx(1, 1)
    lax.fori_loop(0, N_CHUNKS // 2, step, 0)
    put_copy(N_CHUNKS - 1, 1).wait()


def _sc_fuse(ftab, cid, colid, styid):
    mesh = plsc.VectorSubcoreMesh(core_axis_name="c", subcore_axis_name="s")
    fn = functools.partial(
        pl.kernel,
        mesh=mesh,
        out_type=jax.ShapeDtypeStruct((L, B, D), jnp.float32),
        scratch_types=[
            pltpu.VMEM((CHUNK,), jnp.int32),
            pltpu.VMEM((CHUNK,), jnp.int32),
            pltpu.VMEM((CHUNK,), jnp.int32),
            pltpu.VMEM((CHUNK,), jnp.int32),
            pltpu.VMEM((CHUNK,), jnp.int32),
            pltpu.VMEM((CHUNK,), jnp.int32),
            pltpu.VMEM((3 * CHUNK, D), jnp.float32),
            pltpu.VMEM((3 * CHUNK, D), jnp.float32),
            pltpu.SemaphoreType.DMA,
            pltpu.SemaphoreType.DMA,
            pltpu.SemaphoreType.DMA,
            pltpu.SemaphoreType.DMA,
            pltpu.SemaphoreType.DMA,
            pltpu.SemaphoreType.DMA,
        ],
    )(_sc_body)
    return fn(ftab, cid, colid, styid)


def kernel(category_ids, color_ids, style_ids, cat_table, color_table,
           style_table, W, b):
    ftab = _build_fused_table(cat_table, color_table, style_table, W,
                              b.reshape(1, D))
    out_lbd = _sc_fuse(ftab,
                       jnp.transpose(category_ids).astype(jnp.int32),
                       jnp.transpose(color_ids).astype(jnp.int32) + COL_BASE,
                       jnp.transpose(style_ids).astype(jnp.int32) + STY_BASE)
    return jnp.transpose(out_lbd, (1, 0, 2))
